# baseline (speedup 1.0000x reference)
import jax
import jax.numpy as jnp
from jax.experimental import pallas as pl
from jax.experimental.pallas import tpu as pltpu

_R = 512
_C = 256
_CB = 2048
_L = 128


def _cumavg_kernel(x_ref, rhs_ref, out_ref, carry_ref):
    j = pl.program_id(1)

    @pl.when(j == 0)
    def _():
        carry_ref[...] = jnp.zeros_like(carry_ref)

    rhs = rhs_ref[...]
    c = carry_ref[...]
    for k in range(_CB // _C):
        x = x_ref[:, k * _C:(k + 1) * _C]
        hi = x.astype(jnp.bfloat16)
        lo = (x - hi.astype(jnp.float32)).astype(jnp.bfloat16)
        z = jnp.dot(hi, rhs, preferred_element_type=jnp.float32)
        z = z + jnp.dot(lo, rhs, preferred_element_type=jnp.float32)
        base = j * _CB + k * _C
        it = jax.lax.broadcasted_iota(jnp.int32, (1, _L), 1)
        for h in range(_C // _L):
            yh = z[:, h * _L:(h + 1) * _L] + c
            cnt = (it + (base + h * _L + 1)).astype(jnp.float32)
            out_ref[:, k * _C + h * _L:k * _C + (h + 1) * _L] = yh / cnt
        c = c + z[:, _C:_C + _L]
    carry_ref[...] = c


@jax.jit
def kernel(x):
    b, ch, t = x.shape
    rows = b * ch
    xr = x.reshape(rows, t)
    rhs = jnp.concatenate(
        [jnp.triu(jnp.ones((_C, _C), jnp.float32)),
         jnp.ones((_C, _L), jnp.float32)], axis=1).astype(jnp.bfloat16)
    grid = (rows // _R, t // _CB)
    out = pl.pallas_call(
        _cumavg_kernel,
        grid=grid,
        in_specs=[
            pl.BlockSpec((_R, _CB), lambda i, j: (i, j)),
            pl.BlockSpec((_C, _C + _L), lambda i, j: (0, 0)),
        ],
        out_specs=pl.BlockSpec((_R, _CB), lambda i, j: (i, j)),
        out_shape=jax.ShapeDtypeStruct((rows, t), jnp.float32),
        scratch_shapes=[pltpu.VMEM((_R, _L), jnp.float32)],
        compiler_params=pltpu.CompilerParams(
            dimension_semantics=("parallel", "arbitrary"),
        ),
    )(xr, rhs)
    return out.reshape(b, ch, t)


# speedup vs baseline: 1.1467x; 1.1467x over previous
import jax
import jax.numpy as jnp
from jax.experimental import pallas as pl
from jax.experimental.pallas import tpu as pltpu

_R = 4096
_C = 256
_CB = 256


def _cumavg_kernel(x_ref, tri_ref, out_ref, carry_ref):
    j = pl.program_id(1)

    @pl.when(j == 0)
    def _():
        carry_ref[...] = jnp.zeros_like(carry_ref)

    tri = tri_ref[...]
    carry = carry_ref[:, 0:1]
    for k in range(_CB // _C):
        x = x_ref[:, k * _C:(k + 1) * _C]
        hi = x.astype(jnp.bfloat16)
        lo = (x - hi.astype(jnp.float32)).astype(jnp.bfloat16)
        y = jnp.dot(hi, tri, preferred_element_type=jnp.float32)
        y = y + jnp.dot(lo, tri, preferred_element_type=jnp.float32)
        y = y + carry
        carry = y[:, _C - 1:_C]
        it = jax.lax.broadcasted_iota(jnp.int32, (1, _C), 1) + (
            j * _CB + k * _C + 1)
        out_ref[:, k * _C:(k + 1) * _C] = y / it.astype(jnp.float32)
    carry_ref[...] = jnp.broadcast_to(carry, carry_ref.shape)


@jax.jit
def kernel(x):
    b, ch, t = x.shape
    rows = b * ch
    xr = x.reshape(rows, t)
    tri = jnp.triu(jnp.ones((_C, _C), jnp.bfloat16))
    grid = (rows // _R, t // _CB)
    out = pl.pallas_call(
        _cumavg_kernel,
        grid=grid,
        in_specs=[
            pl.BlockSpec((_R, _CB), lambda i, j: (i, j)),
            pl.BlockSpec((_C, _C), lambda i, j: (0, 0)),
        ],
        out_specs=pl.BlockSpec((_R, _CB), lambda i, j: (i, j)),
        out_shape=jax.ShapeDtypeStruct((rows, t), jnp.float32),
        scratch_shapes=[pltpu.VMEM((_R, 128), jnp.float32)],
        compiler_params=pltpu.CompilerParams(
            dimension_semantics=("parallel", "arbitrary"),
        ),
    )(xr, tri)
    return out.reshape(b, ch, t)


# speedup vs baseline: 1.1838x; 1.0324x over previous
import jax
import jax.numpy as jnp
from jax.experimental import pallas as pl
from jax.experimental.pallas import tpu as pltpu

_R = 4096
_C = 256
_CB = 512


def _cumavg_kernel(x_ref, tri_ref, out_ref, carry_ref):
    j = pl.program_id(1)

    @pl.when(j == 0)
    def _():
        carry_ref[...] = jnp.zeros_like(carry_ref)

    tri = tri_ref[...]
    carry = carry_ref[:, 0:1]
    for k in range(_CB // _C):
        x = x_ref[:, k * _C:(k + 1) * _C]
        hi = x.astype(jnp.bfloat16)
        lo = (x - hi.astype(jnp.float32)).astype(jnp.bfloat16)
        y = jnp.dot(hi, tri, preferred_element_type=jnp.float32)
        y = y + jnp.dot(lo, tri, preferred_element_type=jnp.float32)
        y = y + carry
        carry = y[:, _C - 1:_C]
        it = jax.lax.broadcasted_iota(jnp.int32, (1, _C), 1) + (
            j * _CB + k * _C + 1)
        out_ref[:, k * _C:(k + 1) * _C] = y / it.astype(jnp.float32)
    carry_ref[...] = jnp.broadcast_to(carry, carry_ref.shape)


@jax.jit
def kernel(x):
    b, ch, t = x.shape
    rows = b * ch
    xr = x.reshape(rows, t)
    tri = jnp.triu(jnp.ones((_C, _C), jnp.bfloat16))
    grid = (rows // _R, t // _CB)
    out = pl.pallas_call(
        _cumavg_kernel,
        grid=grid,
        in_specs=[
            pl.BlockSpec((_R, _CB), lambda i, j: (i, j)),
            pl.BlockSpec((_C, _C), lambda i, j: (0, 0)),
        ],
        out_specs=pl.BlockSpec((_R, _CB), lambda i, j: (i, j)),
        out_shape=jax.ShapeDtypeStruct((rows, t), jnp.float32),
        scratch_shapes=[pltpu.VMEM((_R, 128), jnp.float32)],
        compiler_params=pltpu.CompilerParams(
            dimension_semantics=("parallel", "arbitrary"),
        ),
    )(xr, tri)
    return out.reshape(b, ch, t)
